# baseline (speedup 1.0000x reference)
import functools

import jax
import jax.numpy as jnp
from jax import lax
from jax.experimental import pallas as pl
from jax.experimental.pallas import tpu as pltpu
from jax.experimental.pallas import tpu_sc as plsc

_STRIDE = 8.0
_IMG_W = 512.0
_NCLS = 80
_NB = 16
_NPIX = 4096
_HALF = _NPIX // 2
_CHUNK = 512
_NCHUNK = _HALF // _CHUNK


def _sc_body(cls_hbm, m_hbm, idx_hbm, buf, mv, iv):
    c = lax.axis_index("c")
    s = lax.axis_index("s")
    wid = s * 2 + c
    img = wid // 2
    p_base = (wid % 2) * _HALF
    lanes = lax.iota(jnp.int32, 16)
    zeros16 = jnp.zeros((16,), jnp.int32)

    for ch in range(_NCHUNK):
        p0 = p_base + ch * _CHUNK
        pltpu.sync_copy(cls_hbm.at[img, pl.ds(p0, _CHUNK), :], buf)
        for g in range(_CHUNK // 16):
            pvec = lanes + (g * 16)

            def step(k, carry):
                m, ix = carry
                v = plsc.load_gather(buf, [pvec, zeros16 + k])
                upd = v > m
                return jnp.where(upd, v, m), jnp.where(upd, k, ix)

            m0 = plsc.load_gather(buf, [pvec, zeros16])
            m, ix = lax.fori_loop(1, _NCLS, step, (m0, zeros16))
            mv[pl.ds(g * 16, 16)] = m
            iv[pl.ds(g * 16, 16)] = ix
        pltpu.sync_copy(mv, m_hbm.at[img, pl.ds(p0, _CHUNK)])
        pltpu.sync_copy(iv, idx_hbm.at[img, pl.ds(p0, _CHUNK)])


_sc_maxargmax = pl.kernel(
    _sc_body,
    out_type=(
        jax.ShapeDtypeStruct((_NB, _NPIX), jnp.float32),
        jax.ShapeDtypeStruct((_NB, _NPIX), jnp.int32),
    ),
    mesh=plsc.VectorSubcoreMesh(core_axis_name="c", subcore_axis_name="s"),
    scratch_types=[
        pltpu.VMEM((_CHUNK, _NCLS), jnp.float32),
        pltpu.VMEM((_CHUNK,), jnp.float32),
        pltpu.VMEM((_CHUNK,), jnp.int32),
    ],
    compiler_params=pltpu.CompilerParams(use_tc_tiling_on_sc=True,
                                         needs_layout_passes=False),
)


def _tc_body(bbox_ref, conf_ref, m_ref, obb_ref, osc_ref):
    b = bbox_ref[0]
    fr = jax.lax.broadcasted_iota(jnp.int32, (128, 128), 0)
    fl = jax.lax.broadcasted_iota(jnp.int32, (128, 128), 1)
    flat = fr * 128 + fl
    pix = flat >> 2
    chan = flat & 3
    xc = (pix & 63).astype(jnp.float32) * _STRIDE + _STRIDE / 2.0
    yc = ((pix >> 6) & 63).astype(jnp.float32) * _STRIDE + _STRIDE / 2.0
    ctr = jnp.where((chan & 1) == 0, xc, yc)
    sgn = jnp.where(chan < 2, -1.0, 1.0)
    e = jnp.clip(ctr + sgn * (jnp.exp(b) * _STRIDE), 0.0, _IMG_W)
    el = pltpu.roll(e, 126, 1)
    er = pltpu.roll(e, 2, 1)
    obb_ref[0] = jnp.where(chan < 2, (e + el) * 0.5, e - er)
    osc_ref[0] = jnp.sqrt(jax.nn.sigmoid(conf_ref[0])
                          * jax.nn.sigmoid(m_ref[0]))


def kernel(bbox, conf, cls):
    nB, nH, nW, _ = bbox.shape
    npix = nH * nW
    cls_r = cls.reshape(nB, npix, _NCLS)
    m2d, idx2d = _sc_maxargmax(cls_r)

    bbox_r = bbox.reshape(nB, 128, 128)
    conf_r = conf.reshape(nB, 32, 128)
    m_r = m2d.reshape(nB, 32, 128)
    obb, osc = pl.pallas_call(
        _tc_body,
        grid=(nB,),
        in_specs=[
            pl.BlockSpec((1, 128, 128), lambda i: (i, 0, 0)),
            pl.BlockSpec((1, 32, 128), lambda i: (i, 0, 0)),
            pl.BlockSpec((1, 32, 128), lambda i: (i, 0, 0)),
        ],
        out_specs=(
            pl.BlockSpec((1, 128, 128), lambda i: (i, 0, 0)),
            pl.BlockSpec((1, 32, 128), lambda i: (i, 0, 0)),
        ),
        out_shape=(
            jax.ShapeDtypeStruct((nB, 128, 128), jnp.float32),
            jax.ShapeDtypeStruct((nB, 32, 128), jnp.float32),
        ),
        compiler_params=pltpu.CompilerParams(
            dimension_semantics=("parallel",)),
    )(bbox_r, conf_r, m_r)
    return (obb.reshape(nB, npix, 4), idx2d, osc.reshape(nB, npix))


# speedup vs baseline: 1.4031x; 1.4031x over previous
import functools

import jax
import jax.numpy as jnp
from jax import lax
from jax.experimental import pallas as pl
from jax.experimental.pallas import tpu as pltpu
from jax.experimental.pallas import tpu_sc as plsc

_STRIDE = 8.0
_IMG_W = 512.0
_NCLS = 80
_NB = 16
_NPIX = 4096
_HALF = _NPIX // 2
_CHUNK = 512
_NCHUNK = _HALF // _CHUNK


def _sc_body(cls_hbm, m_hbm, idx_hbm, buf, mv, iv):
    c = lax.axis_index("c")
    s = lax.axis_index("s")
    wid = s * 2 + c
    img = wid // 2
    p_base = (wid % 2) * _HALF
    lanes = lax.iota(jnp.int32, 16)
    zeros16 = jnp.zeros((16,), jnp.int32)

    def merge(a, b):
        va, ia = a
        vb, ib = b
        t = vb > va
        return jnp.where(t, vb, va), jnp.where(t, ib, ia)

    for ch in range(_NCHUNK):
        p0 = p_base + ch * _CHUNK
        pltpu.sync_copy(cls_hbm.at[img, pl.ds(p0, _CHUNK), :], buf)

        def group_body(g, carry):
            pvec = lanes + g * 16
            blocks = []
            for b0 in range(0, _NCLS, 16):
                nodes = [
                    (plsc.load_gather(buf, [pvec, zeros16 + (b0 + j)]),
                     zeros16 + (b0 + j))
                    for j in range(16)
                ]
                while len(nodes) > 1:
                    nodes = [merge(nodes[i], nodes[i + 1])
                             for i in range(0, len(nodes), 2)]
                blocks.append(nodes[0])
            while len(blocks) > 1:
                nxt = [merge(blocks[i], blocks[i + 1])
                       for i in range(0, len(blocks) - 1, 2)]
                if len(blocks) % 2:
                    nxt.append(blocks[-1])
                blocks = nxt
            m, ix = blocks[0]
            plsc.store_scatter(mv, [pvec], m)
            plsc.store_scatter(iv, [pvec], ix)
            return carry

        lax.fori_loop(0, _CHUNK // 16, group_body, 0)
        pltpu.sync_copy(mv, m_hbm.at[img, pl.ds(p0, _CHUNK)])
        pltpu.sync_copy(iv, idx_hbm.at[img, pl.ds(p0, _CHUNK)])


_sc_maxargmax = pl.kernel(
    _sc_body,
    out_type=(
        jax.ShapeDtypeStruct((_NB, _NPIX), jnp.float32),
        jax.ShapeDtypeStruct((_NB, _NPIX), jnp.int32),
    ),
    mesh=plsc.VectorSubcoreMesh(core_axis_name="c", subcore_axis_name="s"),
    scratch_types=[
        pltpu.VMEM((_CHUNK, _NCLS), jnp.float32),
        pltpu.VMEM((_CHUNK,), jnp.float32),
        pltpu.VMEM((_CHUNK,), jnp.int32),
    ],
    compiler_params=pltpu.CompilerParams(use_tc_tiling_on_sc=True,
                                         needs_layout_passes=False),
)


def _tc_body(bbox_ref, conf_ref, m_ref, obb_ref, osc_ref):
    b = bbox_ref[0]
    fr = jax.lax.broadcasted_iota(jnp.int32, (128, 128), 0)
    fl = jax.lax.broadcasted_iota(jnp.int32, (128, 128), 1)
    flat = fr * 128 + fl
    pix = flat >> 2
    chan = flat & 3
    xc = (pix & 63).astype(jnp.float32) * _STRIDE + _STRIDE / 2.0
    yc = ((pix >> 6) & 63).astype(jnp.float32) * _STRIDE + _STRIDE / 2.0
    ctr = jnp.where((chan & 1) == 0, xc, yc)
    sgn = jnp.where(chan < 2, -1.0, 1.0)
    e = jnp.clip(ctr + sgn * (jnp.exp(b) * _STRIDE), 0.0, _IMG_W)
    el = pltpu.roll(e, 126, 1)
    er = pltpu.roll(e, 2, 1)
    obb_ref[0] = jnp.where(chan < 2, (e + el) * 0.5, e - er)
    osc_ref[0] = jnp.sqrt(jax.nn.sigmoid(conf_ref[0])
                          * jax.nn.sigmoid(m_ref[0]))


def kernel(bbox, conf, cls):
    nB, nH, nW, _ = bbox.shape
    npix = nH * nW
    cls_r = cls.reshape(nB, npix, _NCLS)
    m2d, idx2d = _sc_maxargmax(cls_r)

    bbox_r = bbox.reshape(nB, 128, 128)
    conf_r = conf.reshape(nB, 32, 128)
    m_r = m2d.reshape(nB, 32, 128)
    obb, osc = pl.pallas_call(
        _tc_body,
        grid=(nB,),
        in_specs=[
            pl.BlockSpec((1, 128, 128), lambda i: (i, 0, 0)),
            pl.BlockSpec((1, 32, 128), lambda i: (i, 0, 0)),
            pl.BlockSpec((1, 32, 128), lambda i: (i, 0, 0)),
        ],
        out_specs=(
            pl.BlockSpec((1, 128, 128), lambda i: (i, 0, 0)),
            pl.BlockSpec((1, 32, 128), lambda i: (i, 0, 0)),
        ),
        out_shape=(
            jax.ShapeDtypeStruct((nB, 128, 128), jnp.float32),
            jax.ShapeDtypeStruct((nB, 32, 128), jnp.float32),
        ),
        compiler_params=pltpu.CompilerParams(
            dimension_semantics=("parallel",)),
    )(bbox_r, conf_r, m_r)
    return (obb.reshape(nB, npix, 4), idx2d, osc.reshape(nB, npix))
